# baseline (speedup 1.0000x reference)
import functools

import jax
import jax.numpy as jnp
from jax import lax
from jax.experimental import pallas as pl
from jax.experimental.pallas import tpu as pltpu
from jax.experimental.pallas import tpu_sc as plsc

N = 10000
D = 128
HD = D // 2
E = 320000
NCORE = 2
NSUB = 16
NW = NCORE * NSUB
LG = 128
GP = 80
EP = NW * LG * GP
ROWS_PT = 640
NR = NSUB * ROWS_PT
DW = 16
NBUF = 4
LGA = 128
GPAIR = 2 * (EP // (NW * LGA))
G0 = 48
G1 = GPAIR - G0
RB = 2000

_MESH = dict(core_axis_name="c", subcore_axis_name="s")
_SC_PARAMS = pltpu.CompilerParams(use_tc_tiling_on_sc=False)


def _sc_degree(dst2d):
    @functools.partial(
        pl.kernel,
        out_type=jax.ShapeDtypeStruct((NCORE, NR, DW), jnp.float32),
        mesh=plsc.VectorSubcoreMesh(**_MESH),
        compiler_params=_SC_PARAMS,
        scratch_types=[
            pltpu.VMEM((GP, LG), jnp.int32),
            pltpu.VMEM((LG, DW), jnp.float32),
            pltpu.VMEM((16, DW), jnp.float32),
            pltpu.VMEM_SHARED((NR, DW), jnp.float32),
        ],
    )
    def k(dst_hbm, out_hbm, idx_v, ones_v, z_v, acc):
        c = lax.axis_index("c")
        s = lax.axis_index("s")
        wid = s * NCORE + c

        @pl.loop(0, LG)
        def _(r):
            ones_v[r, :] = jnp.ones((DW,), jnp.float32)

        @pl.loop(0, 16)
        def _(r):
            z_v[r, :] = jnp.zeros((DW,), jnp.float32)

        @pl.loop(0, ROWS_PT, step=16)
        def _(r):
            pltpu.sync_copy(z_v, acc.at[pl.ds(s * ROWS_PT + r, 16)])

        plsc.subcore_barrier()
        pltpu.sync_copy(dst_hbm.at[pl.ds(wid * GP, GP)], idx_v)

        @pl.loop(0, GP)
        def _(j):
            pltpu.sync_copy(ones_v, acc.at[idx_v.at[j]], add=True)

        plsc.subcore_barrier()
        pltpu.sync_copy(acc.at[pl.ds(s * ROWS_PT, ROWS_PT)],
                        out_hbm.at[c, pl.ds(s * ROWS_PT, ROWS_PT)])

    return k(dst2d)


def _sc_aggregate(src2d, dst2d, g_lo, g_hi):
    @functools.partial(
        pl.kernel,
        out_type=jax.ShapeDtypeStruct((2, NCORE, NR, HD), jnp.float32),
        mesh=plsc.VectorSubcoreMesh(**_MESH),
        compiler_params=_SC_PARAMS,
        scratch_types=[
            pltpu.VMEM((max(G0, G1), LGA), jnp.int32),
            pltpu.VMEM((max(G0, G1), LGA), jnp.int32),
            [pltpu.VMEM((LGA, HD), jnp.float32)] * NBUF,
            pltpu.VMEM((16, HD), jnp.float32),
            pltpu.VMEM_SHARED((NR, HD), jnp.float32),
            [pltpu.SemaphoreType.DMA] * NBUF,
            [pltpu.SemaphoreType.DMA] * NBUF,
        ],
    )
    def k(src_hbm, dst_hbm, glo_hbm, ghi_hbm, out_hbm,
          si, di, bufs, z_v, acc, sg, ss):
        c = lax.axis_index("c")
        s = lax.axis_index("s")
        gc = jnp.where(c == 0, G0, G1)
        goff = s * GPAIR + c * G0

        @pl.loop(0, 16)
        def _(r):
            @pl.loop(0, HD, step=16)
            def _(q):
                z_v[r, pl.ds(q, 16)] = jnp.zeros((16,), jnp.float32)

        pltpu.sync_copy(src_hbm.at[pl.ds(goff, max(G0, G1))], si)
        pltpu.sync_copy(dst_hbm.at[pl.ds(goff, max(G0, G1))], di)

        for h, g_hbm in ((0, glo_hbm), (1, ghi_hbm)):
            @pl.loop(0, ROWS_PT, step=16)
            def _(r):
                pltpu.async_copy(z_v, acc.at[pl.ds(s * ROWS_PT + r, 16)], sg[0])

            @pl.loop(0, ROWS_PT, step=16)
            def _(r):
                pltpu.make_async_copy(z_v, acc.at[pl.ds(s * ROWS_PT + r, 16)], sg[0]).wait()

            plsc.subcore_barrier()

            for b in range(NBUF):
                pltpu.async_copy(g_hbm.at[si.at[b]], bufs[b], sg[b])

            @pl.loop(0, gc, step=NBUF)
            def _(j):
                for b in range(NBUF):
                    jj = j + b
                    pltpu.make_async_copy(g_hbm.at[si.at[jj]], bufs[b], sg[b]).wait()
                    pltpu.async_copy(bufs[b], acc.at[di.at[jj]], ss[b], add=True)

                    @pl.when(jj + NBUF < gc)
                    def _():
                        pltpu.make_async_copy(bufs[b], acc.at[di.at[jj]], ss[b]).wait()
                        pltpu.async_copy(g_hbm.at[si.at[jj + NBUF]], bufs[b], sg[b])

            for b in range(NBUF):
                pltpu.make_async_copy(bufs[b], acc.at[di.at[gc - NBUF + b]], ss[b]).wait()

            plsc.subcore_barrier()
            pltpu.sync_copy(acc.at[pl.ds(s * ROWS_PT, ROWS_PT)],
                            out_hbm.at[h, c, pl.ds(s * ROWS_PT, ROWS_PT)])

    return k(src2d, dst2d, g_lo, g_hi)


def _tc_matmul(v, w):
    def body(v_ref, w_ref, o_ref):
        o_ref[...] = jnp.dot(v_ref[...], w_ref[...],
                             preferred_element_type=jnp.float32,
                             precision=lax.Precision.HIGHEST)

    return pl.pallas_call(
        body,
        grid=(N // RB,),
        in_specs=[pl.BlockSpec((RB, D), lambda i: (i, 0)),
                  pl.BlockSpec((D, D), lambda i: (0, 0))],
        out_specs=pl.BlockSpec((RB, D), lambda i: (i, 0)),
        out_shape=jax.ShapeDtypeStruct((N, D), jnp.float32),
    )(v, w)


def _dis_of(d_ref):
    deg = d_ref[0, :, :1] + d_ref[1, :, :1] + 1.0
    return lax.rsqrt(deg)


_HALF_OUT = [jax.ShapeDtypeStruct((N, HD), jnp.float32)] * 2
_HALF_SPECS = [pl.BlockSpec((RB, HD), lambda i: (i, 0))] * 2


def _tc_scale(h, degp):
    def body(h_ref, d_ref, lo_ref, hi_ref):
        g = h_ref[...] * _dis_of(d_ref)
        lo_ref[...] = g[:, :HD]
        hi_ref[...] = g[:, HD:]

    return pl.pallas_call(
        body,
        grid=(N // RB,),
        in_specs=[pl.BlockSpec((RB, D), lambda i: (i, 0)),
                  pl.BlockSpec((NCORE, RB, DW), lambda i: (0, i, 0))],
        out_specs=_HALF_SPECS,
        out_shape=_HALF_OUT,
    )(h, degp)


def _tc_layer2(p, g_lo, g_hi, degp, b1, w2):
    def body(p_ref, glo_ref, ghi_ref, d_ref, b_ref, w_ref, lo_ref, hi_ref):
        dis = _dis_of(d_ref)
        s_lo = p_ref[0, 0] + p_ref[0, 1] + glo_ref[...]
        s_hi = p_ref[1, 0] + p_ref[1, 1] + ghi_ref[...]
        h = jnp.concatenate([s_lo, s_hi], axis=1) * dis + b_ref[...]
        h = jnp.maximum(h, 0.0)
        g = jnp.dot(h, w_ref[...],
                    preferred_element_type=jnp.float32,
                    precision=lax.Precision.HIGHEST) * dis
        lo_ref[...] = g[:, :HD]
        hi_ref[...] = g[:, HD:]

    return pl.pallas_call(
        body,
        grid=(N // RB,),
        in_specs=[pl.BlockSpec((2, NCORE, RB, HD), lambda i: (0, 0, i, 0)),
                  pl.BlockSpec((RB, HD), lambda i: (i, 0)),
                  pl.BlockSpec((RB, HD), lambda i: (i, 0)),
                  pl.BlockSpec((NCORE, RB, DW), lambda i: (0, i, 0)),
                  pl.BlockSpec((1, D), lambda i: (0, 0)),
                  pl.BlockSpec((D, D), lambda i: (0, 0))],
        out_specs=_HALF_SPECS,
        out_shape=_HALF_OUT,
    )(p, g_lo, g_hi, degp, b1, w2)


def _tc_final(p, g_lo, g_hi, degp, b2):
    def body(p_ref, glo_ref, ghi_ref, d_ref, b_ref, o_ref):
        dis = _dis_of(d_ref)
        s_lo = p_ref[0, 0] + p_ref[0, 1] + glo_ref[...]
        s_hi = p_ref[1, 0] + p_ref[1, 1] + ghi_ref[...]
        o_ref[...] = jnp.concatenate([s_lo, s_hi], axis=1) * dis + b_ref[...]

    return pl.pallas_call(
        body,
        grid=(N // RB,),
        in_specs=[pl.BlockSpec((2, NCORE, RB, HD), lambda i: (0, 0, i, 0)),
                  pl.BlockSpec((RB, HD), lambda i: (i, 0)),
                  pl.BlockSpec((RB, HD), lambda i: (i, 0)),
                  pl.BlockSpec((NCORE, RB, DW), lambda i: (0, i, 0)),
                  pl.BlockSpec((1, D), lambda i: (0, 0))],
        out_specs=pl.BlockSpec((RB, D), lambda i: (i, 0)),
        out_shape=jax.ShapeDtypeStruct((N, D), jnp.float32),
    )(p, g_lo, g_hi, degp, b2)


def kernel(x, edge_index, W1, b1, W2, b2):
    src = edge_index[0].astype(jnp.int32)
    dst = edge_index[1].astype(jnp.int32)
    pad = EP - E
    srcp = jnp.concatenate([src, jnp.zeros((pad,), jnp.int32)])
    dstp = jnp.concatenate([dst, jnp.full((pad,), N, jnp.int32)])
    src2d = srcp.reshape(EP // LGA, LGA)
    dst2d = dstp.reshape(EP // LGA, LGA)
    dstd = dstp.reshape(EP // LG, LG)

    degp = _sc_degree(dstd)
    h1 = _tc_matmul(x, W1)
    g1_lo, g1_hi = _tc_scale(h1, degp)
    p1 = _sc_aggregate(src2d, dst2d, g1_lo, g1_hi)
    g2_lo, g2_hi = _tc_layer2(p1, g1_lo, g1_hi, degp, b1.reshape(1, D), W2)
    p2 = _sc_aggregate(src2d, dst2d, g2_lo, g2_hi)
    return _tc_final(p2, g2_lo, g2_hi, degp, b2.reshape(1, D))


# speedup vs baseline: 1.1818x; 1.1818x over previous
import functools

import jax
import jax.numpy as jnp
from jax import lax
from jax.experimental import pallas as pl
from jax.experimental.pallas import tpu as pltpu
from jax.experimental.pallas import tpu_sc as plsc

N = 10000
D = 128
HD = D // 2
E = 320000
NCORE = 2
NSUB = 16
NW = NCORE * NSUB
LG = 128
GP = 80
EP = NW * LG * GP
ROWS_PT = 640
NR = NSUB * ROWS_PT
DW = 16
NBUF = 4
LGA = 128
GPAIR = 2 * (EP // (NW * LGA))
G0 = 112
G1 = GPAIR - G0
RB = 2000

_MESH = dict(core_axis_name="c", subcore_axis_name="s")
_SC_PARAMS = pltpu.CompilerParams(use_tc_tiling_on_sc=False)


def _sc_degree(dst2d):
    @functools.partial(
        pl.kernel,
        out_type=jax.ShapeDtypeStruct((NCORE, NR, DW), jnp.float32),
        mesh=plsc.VectorSubcoreMesh(**_MESH),
        compiler_params=_SC_PARAMS,
        scratch_types=[
            pltpu.VMEM((GP, LG), jnp.int32),
            pltpu.VMEM((LG, DW), jnp.float32),
            pltpu.VMEM((16, DW), jnp.float32),
            pltpu.VMEM_SHARED((NR, DW), jnp.float32),
        ],
    )
    def k(dst_hbm, out_hbm, idx_v, ones_v, z_v, acc):
        c = lax.axis_index("c")
        s = lax.axis_index("s")
        wid = s * NCORE + c

        @pl.loop(0, LG)
        def _(r):
            ones_v[r, :] = jnp.ones((DW,), jnp.float32)

        @pl.loop(0, 16)
        def _(r):
            z_v[r, :] = jnp.zeros((DW,), jnp.float32)

        @pl.loop(0, ROWS_PT, step=16)
        def _(r):
            pltpu.sync_copy(z_v, acc.at[pl.ds(s * ROWS_PT + r, 16)])

        plsc.subcore_barrier()
        pltpu.sync_copy(dst_hbm.at[pl.ds(wid * GP, GP)], idx_v)

        @pl.loop(0, GP)
        def _(j):
            pltpu.sync_copy(ones_v, acc.at[idx_v.at[j]], add=True)

        plsc.subcore_barrier()
        pltpu.sync_copy(acc.at[pl.ds(s * ROWS_PT, ROWS_PT)],
                        out_hbm.at[c, pl.ds(s * ROWS_PT, ROWS_PT)])

    return k(dst2d)


def _sc_aggregate(src2d, dst2d, g_lo, g_hi):
    @functools.partial(
        pl.kernel,
        out_type=jax.ShapeDtypeStruct((2, NCORE, NR, HD), jnp.float32),
        mesh=plsc.VectorSubcoreMesh(**_MESH),
        compiler_params=_SC_PARAMS,
        scratch_types=[
            pltpu.VMEM((max(G0, G1), LGA), jnp.int32),
            pltpu.VMEM((max(G0, G1), LGA), jnp.int32),
            [pltpu.VMEM((LGA, HD), jnp.float32)] * NBUF,
            pltpu.VMEM((16, HD), jnp.float32),
            pltpu.VMEM_SHARED((NR, HD), jnp.float32),
            [pltpu.SemaphoreType.DMA] * NBUF,
            [pltpu.SemaphoreType.DMA] * NBUF,
        ],
    )
    def k(src_hbm, dst_hbm, glo_hbm, ghi_hbm, out_hbm,
          si, di, bufs, z_v, acc, sg, ss):
        c = lax.axis_index("c")
        s = lax.axis_index("s")
        gc = jnp.where(c == 0, G0, G1)
        goff = s * GPAIR + c * G0

        @pl.loop(0, 16)
        def _(r):
            @pl.loop(0, HD, step=16)
            def _(q):
                z_v[r, pl.ds(q, 16)] = jnp.zeros((16,), jnp.float32)

        pltpu.sync_copy(src_hbm.at[pl.ds(goff, max(G0, G1))], si)
        pltpu.sync_copy(dst_hbm.at[pl.ds(goff, max(G0, G1))], di)

        for h, g_hbm in ((0, glo_hbm), (1, ghi_hbm)):
            @pl.loop(0, ROWS_PT, step=16)
            def _(r):
                pltpu.async_copy(z_v, acc.at[pl.ds(s * ROWS_PT + r, 16)], sg[0])

            @pl.loop(0, ROWS_PT, step=16)
            def _(r):
                pltpu.make_async_copy(z_v, acc.at[pl.ds(s * ROWS_PT + r, 16)], sg[0]).wait()

            plsc.subcore_barrier()

            for b in range(NBUF):
                pltpu.async_copy(g_hbm.at[si.at[b]], bufs[b], sg[b])

            @pl.loop(0, gc, step=NBUF)
            def _(j):
                for b in range(NBUF):
                    jj = j + b
                    pltpu.make_async_copy(g_hbm.at[si.at[jj]], bufs[b], sg[b]).wait()
                    pltpu.async_copy(bufs[b], acc.at[di.at[jj]], ss[b], add=True)

                    @pl.when(jj + NBUF < gc)
                    def _():
                        pltpu.make_async_copy(bufs[b], acc.at[di.at[jj]], ss[b]).wait()
                        pltpu.async_copy(g_hbm.at[si.at[jj + NBUF]], bufs[b], sg[b])

            for b in range(NBUF):
                pltpu.make_async_copy(bufs[b], acc.at[di.at[gc - NBUF + b]], ss[b]).wait()

            plsc.subcore_barrier()
            pltpu.sync_copy(acc.at[pl.ds(s * ROWS_PT, ROWS_PT)],
                            out_hbm.at[h, c, pl.ds(s * ROWS_PT, ROWS_PT)])

    return k(src2d, dst2d, g_lo, g_hi)


def _tc_matmul(v, w):
    def body(v_ref, w_ref, o_ref):
        o_ref[...] = jnp.dot(v_ref[...], w_ref[...],
                             preferred_element_type=jnp.float32,
                             precision=lax.Precision.HIGHEST)

    return pl.pallas_call(
        body,
        grid=(N // RB,),
        in_specs=[pl.BlockSpec((RB, D), lambda i: (i, 0)),
                  pl.BlockSpec((D, D), lambda i: (0, 0))],
        out_specs=pl.BlockSpec((RB, D), lambda i: (i, 0)),
        out_shape=jax.ShapeDtypeStruct((N, D), jnp.float32),
    )(v, w)


def _dis_of(d_ref):
    deg = d_ref[0, :, :1] + d_ref[1, :, :1] + 1.0
    return lax.rsqrt(deg)


_HALF_OUT = [jax.ShapeDtypeStruct((N, HD), jnp.float32)] * 2
_HALF_SPECS = [pl.BlockSpec((RB, HD), lambda i: (i, 0))] * 2


def _tc_scale(h, degp):
    def body(h_ref, d_ref, lo_ref, hi_ref):
        g = h_ref[...] * _dis_of(d_ref)
        lo_ref[...] = g[:, :HD]
        hi_ref[...] = g[:, HD:]

    return pl.pallas_call(
        body,
        grid=(N // RB,),
        in_specs=[pl.BlockSpec((RB, D), lambda i: (i, 0)),
                  pl.BlockSpec((NCORE, RB, DW), lambda i: (0, i, 0))],
        out_specs=_HALF_SPECS,
        out_shape=_HALF_OUT,
    )(h, degp)


def _tc_layer2(p, g_lo, g_hi, degp, b1, w2):
    def body(p_ref, glo_ref, ghi_ref, d_ref, b_ref, w_ref, lo_ref, hi_ref):
        dis = _dis_of(d_ref)
        s_lo = p_ref[0, 0] + p_ref[0, 1] + glo_ref[...]
        s_hi = p_ref[1, 0] + p_ref[1, 1] + ghi_ref[...]
        h = jnp.concatenate([s_lo, s_hi], axis=1) * dis + b_ref[...]
        h = jnp.maximum(h, 0.0)
        g = jnp.dot(h, w_ref[...],
                    preferred_element_type=jnp.float32,
                    precision=lax.Precision.HIGHEST) * dis
        lo_ref[...] = g[:, :HD]
        hi_ref[...] = g[:, HD:]

    return pl.pallas_call(
        body,
        grid=(N // RB,),
        in_specs=[pl.BlockSpec((2, NCORE, RB, HD), lambda i: (0, 0, i, 0)),
                  pl.BlockSpec((RB, HD), lambda i: (i, 0)),
                  pl.BlockSpec((RB, HD), lambda i: (i, 0)),
                  pl.BlockSpec((NCORE, RB, DW), lambda i: (0, i, 0)),
                  pl.BlockSpec((1, D), lambda i: (0, 0)),
                  pl.BlockSpec((D, D), lambda i: (0, 0))],
        out_specs=_HALF_SPECS,
        out_shape=_HALF_OUT,
    )(p, g_lo, g_hi, degp, b1, w2)


def _tc_final(p, g_lo, g_hi, degp, b2):
    def body(p_ref, glo_ref, ghi_ref, d_ref, b_ref, o_ref):
        dis = _dis_of(d_ref)
        s_lo = p_ref[0, 0] + p_ref[0, 1] + glo_ref[...]
        s_hi = p_ref[1, 0] + p_ref[1, 1] + ghi_ref[...]
        o_ref[...] = jnp.concatenate([s_lo, s_hi], axis=1) * dis + b_ref[...]

    return pl.pallas_call(
        body,
        grid=(N // RB,),
        in_specs=[pl.BlockSpec((2, NCORE, RB, HD), lambda i: (0, 0, i, 0)),
                  pl.BlockSpec((RB, HD), lambda i: (i, 0)),
                  pl.BlockSpec((RB, HD), lambda i: (i, 0)),
                  pl.BlockSpec((NCORE, RB, DW), lambda i: (0, i, 0)),
                  pl.BlockSpec((1, D), lambda i: (0, 0))],
        out_specs=pl.BlockSpec((RB, D), lambda i: (i, 0)),
        out_shape=jax.ShapeDtypeStruct((N, D), jnp.float32),
    )(p, g_lo, g_hi, degp, b2)


def kernel(x, edge_index, W1, b1, W2, b2):
    src = edge_index[0].astype(jnp.int32)
    dst = edge_index[1].astype(jnp.int32)
    pad = EP - E
    srcp = jnp.concatenate([src, jnp.zeros((pad,), jnp.int32)])
    dstp = jnp.concatenate([dst, jnp.full((pad,), N, jnp.int32)])
    src2d = srcp.reshape(EP // LGA, LGA)
    dst2d = dstp.reshape(EP // LGA, LGA)
    dstd = dstp.reshape(EP // LG, LG)

    degp = _sc_degree(dstd)
    h1 = _tc_matmul(x, W1)
    g1_lo, g1_hi = _tc_scale(h1, degp)
    p1 = _sc_aggregate(src2d, dst2d, g1_lo, g1_hi)
    g2_lo, g2_hi = _tc_layer2(p1, g1_lo, g1_hi, degp, b1.reshape(1, D), W2)
    p2 = _sc_aggregate(src2d, dst2d, g2_lo, g2_hi)
    return _tc_final(p2, g2_lo, g2_hi, degp, b2.reshape(1, D))
